# baseline (speedup 1.0000x reference)
import functools

import jax
import jax.numpy as jnp
from jax import lax
from jax.experimental import pallas as pl
from jax.experimental.pallas import tpu as pltpu
from jax.experimental.pallas import tpu_sc as plsc

N = 10000
D = 128
NC = 2
NS = 16
NW = NC * NS
CHUNK = 64
ACC_ROWS = 10240
EPS = 1e-5


def _agg_body(ch_per_w, xw_hbm, src_hbm, dst_hbm, out_hbm,
              src_v, dst_v, rows, zrow_v, acc_sh, sem_g, sem_s):
    cid = lax.axis_index("c")
    sid = lax.axis_index("s")
    wid = cid * NS + sid

    for r in range(8):
        for c in range(D // 16):
            zrow_v[r, pl.ds(c * 16, 16)] = jnp.zeros((16,), jnp.float32)
    rows_per = ACC_ROWS // NS

    def zbody(k, carry):
        pltpu.sync_copy(zrow_v, acc_sh.at[pl.ds(sid * rows_per + k * 8, 8)])
        return carry
    lax.fori_loop(0, rows_per // 8, zbody, 0)

    plsc.subcore_barrier()

    def start_g(b, j):
        pltpu.async_copy(xw_hbm.at[src_v.at[j]], rows[b], sem_g[b])

    def wait_g(b):
        pltpu.make_async_copy(xw_hbm.at[src_v.at[0]], rows[b], sem_g[b]).wait()

    def start_s(b, j):
        pltpu.async_copy(rows[b], acc_sh.at[dst_v.at[j]], sem_s[b], add=True)

    def wait_s(b):
        pltpu.make_async_copy(
            rows[b], acc_sh.at[dst_v.at[0]], sem_s[b]).wait()

    ph_ch = ch_per_w // 4
    assert ph_ch % 4 == 0 and ph_ch >= 8
    for p in range(4):
        pltpu.sync_copy(src_hbm.at[wid].at[pl.ds(p * ph_ch, ph_ch)], src_v)
        pltpu.sync_copy(dst_hbm.at[wid].at[pl.ds(p * ph_ch, ph_ch)], dst_v)
        start_g(0, 0)
        start_g(1, 1)
        start_g(2, 2)
        wait_g(0)
        start_s(0, 0)
        start_g(3, 3)
        wait_g(1)
        start_s(1, 1)

        def body(q, carry):
            cb = 4 * q + 2
            for u in range(4):
                c = cb + u
                wait_s(u)
                start_g(u, c + 2)
                wait_g((u + 2) % 4)
                start_s((u + 2) % 4, c)
            return carry
        lax.fori_loop(0, (ph_ch - 4) // 4, body, 0)
        wait_s(0)
        wait_g(2)
        start_s(2, ph_ch - 2)
        wait_s(1)
        wait_g(3)
        start_s(3, ph_ch - 1)
        wait_s(2)
        wait_s(3)

    plsc.subcore_barrier()
    exp = ACC_ROWS // NS
    pltpu.sync_copy(acc_sh.at[pl.ds(sid * exp, exp)],
                    out_hbm.at[cid].at[pl.ds(sid * exp, exp)])


@functools.partial(jax.jit, static_argnums=(3,))
def _sc_aggregate(xw, src_t, dst_t, ch_per_w):
    mesh = plsc.VectorSubcoreMesh(core_axis_name="c", subcore_axis_name="s")
    return pl.kernel(
        functools.partial(_agg_body, ch_per_w),
        out_type=jax.ShapeDtypeStruct((NC, ACC_ROWS, D), jnp.float32),
        mesh=mesh,
        scratch_types=[
            pltpu.VMEM((ch_per_w // 4, CHUNK), jnp.int32),
            pltpu.VMEM((ch_per_w // 4, CHUNK), jnp.int32),
            tuple(pltpu.VMEM((CHUNK, D), jnp.float32) for _ in range(4)),
            pltpu.VMEM((8, D), jnp.float32),
            pltpu.VMEM_SHARED((ACC_ROWS, D), jnp.float32),
            tuple(pltpu.SemaphoreType.DMA for _ in range(4)),
            tuple(pltpu.SemaphoreType.DMA for _ in range(4)),
        ],
    )(xw, src_t, dst_t)


def _mm_body(x_ref, w_ref, o_ref):
    o_ref[...] = jnp.dot(x_ref[...], w_ref[...],
                         preferred_element_type=jnp.float32)


def _norm_mm_body(p_ref, x_ref, w_ref, b_ref, a_ref, g_ref, be_ref,
                  h_ref, xw_ref):
    s = p_ref[0][:N] + p_ref[1][:N] + b_ref[...]
    mean = jnp.mean(s, axis=0, keepdims=True)
    o = s - a_ref[...] * mean
    var = jnp.mean(o * o, axis=0, keepdims=True)
    o = o / jnp.sqrt(var + EPS)
    o = g_ref[...] * o + be_ref[...]
    h = jnp.maximum(o, 0.0) + x_ref[...]
    h_ref[...] = h
    xw_ref[...] = jnp.dot(h, w_ref[...], preferred_element_type=jnp.float32)


def _norm_body(p_ref, x_ref, b_ref, a_ref, g_ref, be_ref, h_ref):
    s = p_ref[0][:N] + p_ref[1][:N] + b_ref[...]
    mean = jnp.mean(s, axis=0, keepdims=True)
    o = s - a_ref[...] * mean
    var = jnp.mean(o * o, axis=0, keepdims=True)
    o = o / jnp.sqrt(var + EPS)
    o = g_ref[...] * o + be_ref[...]
    h_ref[...] = jnp.maximum(o, 0.0) + x_ref[...]


def kernel(x, edge_index, W1, b1, a1, g1, be1, W2, b2, a2, g2, be2):
    e = edge_index.shape[1]
    ch_per_w = -(-e // (NW * CHUNK))
    ch_per_w = -(-ch_per_w // 16) * 16
    e_pad = NW * ch_per_w * CHUNK
    src = edge_index[0].astype(jnp.int32)
    dst = edge_index[1].astype(jnp.int32)
    pad = e_pad - e
    pad_iota = jnp.arange(pad, dtype=jnp.int32)
    src_t = jnp.concatenate([src, pad_iota % jnp.int32(N)])
    dst_t = jnp.concatenate([dst, N + pad_iota % jnp.int32(ACC_ROWS - N)])
    src_t = src_t.reshape(NW, ch_per_w, CHUNK)
    dst_t = dst_t.reshape(NW, ch_per_w, CHUNK)

    r1 = (jnp.reshape(b1, (1, D)), jnp.reshape(a1, (1, D)),
          jnp.reshape(g1, (1, D)), jnp.reshape(be1, (1, D)))
    r2 = (jnp.reshape(b2, (1, D)), jnp.reshape(a2, (1, D)),
          jnp.reshape(g2, (1, D)), jnp.reshape(be2, (1, D)))

    xw1 = pl.pallas_call(
        _mm_body,
        out_shape=jax.ShapeDtypeStruct((N, D), jnp.float32),
    )(x, W1)

    p1 = _sc_aggregate(xw1, src_t, dst_t, ch_per_w)

    h, xw2 = pl.pallas_call(
        _norm_mm_body,
        out_shape=(jax.ShapeDtypeStruct((N, D), jnp.float32),
                   jax.ShapeDtypeStruct((N, D), jnp.float32)),
    )(p1, x, W2, *r1)

    p2 = _sc_aggregate(xw2, src_t, dst_t, ch_per_w)

    out = pl.pallas_call(
        _norm_body,
        out_shape=jax.ShapeDtypeStruct((N, D), jnp.float32),
    )(p2, h, *r2)
    return out


# speedup vs baseline: 1.0423x; 1.0423x over previous
import functools

import jax
import jax.numpy as jnp
from jax import lax
from jax.experimental import pallas as pl
from jax.experimental.pallas import tpu as pltpu
from jax.experimental.pallas import tpu_sc as plsc

N = 10000
D = 128
NC = 2
NS = 16
NW = NC * NS
CHUNK = 128
ACC_ROWS = 10240
EPS = 1e-5


def _agg_body(ch_per_w, xw_hbm, src_hbm, dst_hbm, out_hbm,
              src_v, dst_v, rows_a, rows_b, zrow_v, acc_sh, sem_a, sem_b):
    cid = lax.axis_index("c")
    sid = lax.axis_index("s")
    wid = cid * NS + sid
    ph_ch = ch_per_w // 2
    last = ph_ch - 1

    pltpu.sync_copy(src_hbm.at[wid].at[pl.ds(0, ph_ch)], src_v)
    pltpu.sync_copy(dst_hbm.at[wid].at[pl.ds(0, ph_ch)], dst_v)
    pltpu.async_copy(xw_hbm.at[src_v.at[0]], rows_a, sem_a)
    pltpu.async_copy(xw_hbm.at[src_v.at[jnp.minimum(1, last)]], rows_b, sem_b)

    for r in range(32):
        for c in range(D // 16):
            zrow_v[r, pl.ds(c * 16, 16)] = jnp.zeros((16,), jnp.float32)
    rows_per = ACC_ROWS // NS

    def zbody(k, carry):
        pltpu.sync_copy(zrow_v, acc_sh.at[pl.ds(sid * rows_per + k * 32, 32)])
        return carry
    lax.fori_loop(0, rows_per // 32, zbody, 0)

    plsc.subcore_barrier()

    for p in range(2):
        if p > 0:
            pltpu.sync_copy(src_hbm.at[wid].at[pl.ds(p * ph_ch, ph_ch)], src_v)
            pltpu.sync_copy(dst_hbm.at[wid].at[pl.ds(p * ph_ch, ph_ch)], dst_v)
            pltpu.async_copy(xw_hbm.at[src_v.at[0]], rows_a, sem_a)
            pltpu.async_copy(
                xw_hbm.at[src_v.at[jnp.minimum(1, last)]], rows_b, sem_b)

        def body(i, carry):
            j = 2 * i
            pltpu.make_async_copy(xw_hbm.at[src_v.at[0]], rows_a, sem_a).wait()
            pltpu.sync_copy(rows_a, acc_sh.at[dst_v.at[j]], add=True)
            pltpu.async_copy(
                xw_hbm.at[src_v.at[jnp.minimum(j + 2, last)]], rows_a, sem_a)
            pltpu.make_async_copy(xw_hbm.at[src_v.at[0]], rows_b, sem_b).wait()
            pltpu.sync_copy(rows_b, acc_sh.at[dst_v.at[j + 1]], add=True)
            pltpu.async_copy(
                xw_hbm.at[src_v.at[jnp.minimum(j + 3, last)]], rows_b, sem_b)
            return carry
        lax.fori_loop(0, ph_ch // 2, body, 0)
        pltpu.make_async_copy(xw_hbm.at[src_v.at[0]], rows_a, sem_a).wait()
        pltpu.make_async_copy(xw_hbm.at[src_v.at[0]], rows_b, sem_b).wait()

    plsc.subcore_barrier()
    exp = ACC_ROWS // NS
    pltpu.sync_copy(acc_sh.at[pl.ds(sid * exp, exp)],
                    out_hbm.at[cid].at[pl.ds(sid * exp, exp)])


@functools.partial(jax.jit, static_argnums=(3,))
def _sc_aggregate(xw, src_t, dst_t, ch_per_w):
    mesh = plsc.VectorSubcoreMesh(core_axis_name="c", subcore_axis_name="s")
    return pl.kernel(
        functools.partial(_agg_body, ch_per_w),
        out_type=jax.ShapeDtypeStruct((NC, ACC_ROWS, D), jnp.float32),
        mesh=mesh,
        scratch_types=[
            pltpu.VMEM((ch_per_w // 2, CHUNK), jnp.int32),
            pltpu.VMEM((ch_per_w // 2, CHUNK), jnp.int32),
            pltpu.VMEM((CHUNK, D), jnp.float32),
            pltpu.VMEM((CHUNK, D), jnp.float32),
            pltpu.VMEM((32, D), jnp.float32),
            pltpu.VMEM_SHARED((ACC_ROWS, D), jnp.float32),
            pltpu.SemaphoreType.DMA,
            pltpu.SemaphoreType.DMA,
        ],
    )(xw, src_t, dst_t)


def _mm_body(x_ref, w_ref, o_ref):
    o_ref[...] = jnp.dot(x_ref[...], w_ref[...],
                         preferred_element_type=jnp.float32)


def _norm_mm_body(p_ref, x_ref, w_ref, b_ref, a_ref, g_ref, be_ref,
                  h_ref, xw_ref):
    s = p_ref[0][:N] + p_ref[1][:N] + b_ref[...]
    mean = jnp.mean(s, axis=0, keepdims=True)
    o = s - a_ref[...] * mean
    var = jnp.mean(o * o, axis=0, keepdims=True)
    o = o / jnp.sqrt(var + EPS)
    o = g_ref[...] * o + be_ref[...]
    h = jnp.maximum(o, 0.0) + x_ref[...]
    h_ref[...] = h
    xw_ref[...] = jnp.dot(h, w_ref[...], preferred_element_type=jnp.float32)


def _norm_body(p_ref, x_ref, b_ref, a_ref, g_ref, be_ref, h_ref):
    s = p_ref[0][:N] + p_ref[1][:N] + b_ref[...]
    mean = jnp.mean(s, axis=0, keepdims=True)
    o = s - a_ref[...] * mean
    var = jnp.mean(o * o, axis=0, keepdims=True)
    o = o / jnp.sqrt(var + EPS)
    o = g_ref[...] * o + be_ref[...]
    h_ref[...] = jnp.maximum(o, 0.0) + x_ref[...]


def kernel(x, edge_index, W1, b1, a1, g1, be1, W2, b2, a2, g2, be2):
    e = edge_index.shape[1]
    ch_per_w = -(-e // (NW * CHUNK))
    ch_per_w = -(-ch_per_w // 16) * 16
    e_pad = NW * ch_per_w * CHUNK
    src = edge_index[0].astype(jnp.int32)
    dst = edge_index[1].astype(jnp.int32)
    pad = e_pad - e
    pad_iota = jnp.arange(pad, dtype=jnp.int32)
    src_t = jnp.concatenate([src, pad_iota % jnp.int32(N)])
    dst_t = jnp.concatenate([dst, N + pad_iota % jnp.int32(ACC_ROWS - N)])
    src_t = src_t.reshape(NW, ch_per_w, CHUNK)
    dst_t = dst_t.reshape(NW, ch_per_w, CHUNK)

    r1 = (jnp.reshape(b1, (1, D)), jnp.reshape(a1, (1, D)),
          jnp.reshape(g1, (1, D)), jnp.reshape(be1, (1, D)))
    r2 = (jnp.reshape(b2, (1, D)), jnp.reshape(a2, (1, D)),
          jnp.reshape(g2, (1, D)), jnp.reshape(be2, (1, D)))

    xw1 = pl.pallas_call(
        _mm_body,
        out_shape=jax.ShapeDtypeStruct((N, D), jnp.float32),
    )(x, W1)

    p1 = _sc_aggregate(xw1, src_t, dst_t, ch_per_w)

    h, xw2 = pl.pallas_call(
        _norm_mm_body,
        out_shape=(jax.ShapeDtypeStruct((N, D), jnp.float32),
                   jax.ShapeDtypeStruct((N, D), jnp.float32)),
    )(p1, x, W2, *r1)

    p2 = _sc_aggregate(xw2, src_t, dst_t, ch_per_w)

    out = pl.pallas_call(
        _norm_body,
        out_shape=jax.ShapeDtypeStruct((N, D), jnp.float32),
    )(p2, h, *r2)
    return out


# speedup vs baseline: 1.0499x; 1.0073x over previous
import functools

import jax
import jax.numpy as jnp
from jax import lax
from jax.experimental import pallas as pl
from jax.experimental.pallas import tpu as pltpu
from jax.experimental.pallas import tpu_sc as plsc

N = 10000
D = 128
NC = 2
NS = 16
NW = NC * NS
CHUNK = 128
ACC_ROWS = 10240
EPS = 1e-5


def _agg_body(ch_per_w, xw_hbm, src_hbm, dst_hbm, out_hbm,
              src_v, dst_v, rows_a, rows_b, zrow_v, acc_sh,
              sem_a, sem_b, sem_z):
    cid = lax.axis_index("c")
    sid = lax.axis_index("s")
    wid = cid * NS + sid
    ph_ch = ch_per_w // 2
    last = ph_ch - 1

    pltpu.sync_copy(src_hbm.at[wid].at[pl.ds(0, ph_ch)], src_v)
    pltpu.sync_copy(dst_hbm.at[wid].at[pl.ds(0, ph_ch)], dst_v)
    pltpu.async_copy(xw_hbm.at[src_v.at[0]], rows_a, sem_a)
    pltpu.async_copy(xw_hbm.at[src_v.at[1]], rows_b, sem_b)

    for r in range(32):
        for c in range(D // 16):
            zrow_v[r, pl.ds(c * 16, 16)] = jnp.zeros((16,), jnp.float32)
    rows_per = ACC_ROWS // NS
    nz = rows_per // 32

    def zbody(k, carry):
        pltpu.async_copy(
            zrow_v, acc_sh.at[pl.ds(sid * rows_per + k * 32, 32)], sem_z)
        return carry
    lax.fori_loop(0, nz, zbody, 0)

    def zdrain(k, carry):
        pltpu.make_async_copy(
            zrow_v, acc_sh.at[pl.ds(sid * rows_per, 32)], sem_z).wait()
        return carry
    lax.fori_loop(0, nz, zdrain, 0)

    plsc.subcore_barrier()

    for p in range(2):
        if p > 0:
            pltpu.sync_copy(src_hbm.at[wid].at[pl.ds(p * ph_ch, ph_ch)], src_v)
            pltpu.sync_copy(dst_hbm.at[wid].at[pl.ds(p * ph_ch, ph_ch)], dst_v)
            pltpu.async_copy(xw_hbm.at[src_v.at[0]], rows_a, sem_a)
            pltpu.async_copy(xw_hbm.at[src_v.at[1]], rows_b, sem_b)

        def body(i, carry):
            j = 2 * i
            pltpu.make_async_copy(xw_hbm.at[src_v.at[0]], rows_a, sem_a).wait()
            pltpu.sync_copy(rows_a, acc_sh.at[dst_v.at[j]], add=True)
            pltpu.async_copy(
                xw_hbm.at[src_v.at[jnp.minimum(j + 2, last)]], rows_a, sem_a)
            pltpu.make_async_copy(xw_hbm.at[src_v.at[0]], rows_b, sem_b).wait()
            pltpu.sync_copy(rows_b, acc_sh.at[dst_v.at[j + 1]], add=True)
            pltpu.async_copy(
                xw_hbm.at[src_v.at[jnp.minimum(j + 3, last)]], rows_b, sem_b)
            return carry
        lax.fori_loop(0, ph_ch // 2, body, 0)
        pltpu.make_async_copy(xw_hbm.at[src_v.at[0]], rows_a, sem_a).wait()
        pltpu.make_async_copy(xw_hbm.at[src_v.at[0]], rows_b, sem_b).wait()

    plsc.subcore_barrier()
    exp = ACC_ROWS // NS
    pltpu.sync_copy(acc_sh.at[pl.ds(sid * exp, exp)],
                    out_hbm.at[cid].at[pl.ds(sid * exp, exp)])


@functools.partial(jax.jit, static_argnums=(3,))
def _sc_aggregate(xw, src_t, dst_t, ch_per_w):
    mesh = plsc.VectorSubcoreMesh(core_axis_name="c", subcore_axis_name="s")
    return pl.kernel(
        functools.partial(_agg_body, ch_per_w),
        out_type=jax.ShapeDtypeStruct((NC, ACC_ROWS, D), jnp.float32),
        mesh=mesh,
        scratch_types=[
            pltpu.VMEM((ch_per_w // 2, CHUNK), jnp.int32),
            pltpu.VMEM((ch_per_w // 2, CHUNK), jnp.int32),
            pltpu.VMEM((CHUNK, D), jnp.float32),
            pltpu.VMEM((CHUNK, D), jnp.float32),
            pltpu.VMEM((32, D), jnp.float32),
            pltpu.VMEM_SHARED((ACC_ROWS, D), jnp.float32),
            pltpu.SemaphoreType.DMA,
            pltpu.SemaphoreType.DMA,
            pltpu.SemaphoreType.DMA,
        ],
    )(xw, src_t, dst_t)


def _mm_body(x_ref, w_ref, o_ref):
    o_ref[...] = jnp.dot(x_ref[...], w_ref[...],
                         preferred_element_type=jnp.float32)


def _norm_mm_body(p_ref, x_ref, w_ref, b_ref, a_ref, g_ref, be_ref,
                  h_ref, xw_ref):
    s = p_ref[0][:N] + p_ref[1][:N] + b_ref[...]
    mean = jnp.mean(s, axis=0, keepdims=True)
    o = s - a_ref[...] * mean
    var = jnp.mean(o * o, axis=0, keepdims=True)
    o = o / jnp.sqrt(var + EPS)
    o = g_ref[...] * o + be_ref[...]
    h = jnp.maximum(o, 0.0) + x_ref[...]
    h_ref[...] = h
    xw_ref[...] = jnp.dot(h, w_ref[...], preferred_element_type=jnp.float32)


def _norm_body(p_ref, x_ref, b_ref, a_ref, g_ref, be_ref, h_ref):
    s = p_ref[0][:N] + p_ref[1][:N] + b_ref[...]
    mean = jnp.mean(s, axis=0, keepdims=True)
    o = s - a_ref[...] * mean
    var = jnp.mean(o * o, axis=0, keepdims=True)
    o = o / jnp.sqrt(var + EPS)
    o = g_ref[...] * o + be_ref[...]
    h_ref[...] = jnp.maximum(o, 0.0) + x_ref[...]


def kernel(x, edge_index, W1, b1, a1, g1, be1, W2, b2, a2, g2, be2):
    e = edge_index.shape[1]
    ch_per_w = -(-e // (NW * CHUNK))
    ch_per_w = -(-ch_per_w // 16) * 16
    e_pad = NW * ch_per_w * CHUNK
    src = edge_index[0].astype(jnp.int32)
    dst = edge_index[1].astype(jnp.int32)
    pad = e_pad - e
    pad_iota = jnp.arange(pad, dtype=jnp.int32)
    src_t = jnp.concatenate([src, pad_iota % jnp.int32(N)])
    dst_t = jnp.concatenate([dst, N + pad_iota % jnp.int32(ACC_ROWS - N)])
    src_t = src_t.reshape(NW, ch_per_w, CHUNK)
    dst_t = dst_t.reshape(NW, ch_per_w, CHUNK)

    r1 = (jnp.reshape(b1, (1, D)), jnp.reshape(a1, (1, D)),
          jnp.reshape(g1, (1, D)), jnp.reshape(be1, (1, D)))
    r2 = (jnp.reshape(b2, (1, D)), jnp.reshape(a2, (1, D)),
          jnp.reshape(g2, (1, D)), jnp.reshape(be2, (1, D)))

    xw1 = pl.pallas_call(
        _mm_body,
        out_shape=jax.ShapeDtypeStruct((N, D), jnp.float32),
    )(x, W1)

    p1 = _sc_aggregate(xw1, src_t, dst_t, ch_per_w)

    h, xw2 = pl.pallas_call(
        _norm_mm_body,
        out_shape=(jax.ShapeDtypeStruct((N, D), jnp.float32),
                   jax.ShapeDtypeStruct((N, D), jnp.float32)),
    )(p1, x, W2, *r1)

    p2 = _sc_aggregate(xw2, src_t, dst_t, ch_per_w)

    out = pl.pallas_call(
        _norm_body,
        out_shape=jax.ShapeDtypeStruct((N, D), jnp.float32),
    )(p2, h, *r2)
    return out
